# baseline (speedup 1.0000x reference)
import functools

import jax
import jax.numpy as jnp
from jax import lax
from jax.experimental import pallas as pl
from jax.experimental.pallas import tpu as pltpu
from jax.experimental.pallas import tpu_sc as plsc

B = 16384
D = 64
H1 = 128
H2 = 64

_NC = 2
_NS = 16
_NW = _NC * _NS
_BPW = B // _NW
_CH = 128
_NCH = _BPW // _CH

_BT = 2048


def _gather_body(uid_hbm, mid_hbm, uemb_hbm, memb_hbm, u_out, m_out,
                 uidx_v, midx_v, urows_v, mrows_v, sem):
    wid = lax.axis_index("s") * _NC + lax.axis_index("c")
    base = wid * _BPW
    pltpu.sync_copy(uid_hbm.at[pl.ds(base, _BPW)], uidx_v)
    pltpu.sync_copy(mid_hbm.at[pl.ds(base, _BPW)], midx_v)
    copies = []
    for ci in range(_NCH):
        sl = pl.ds(ci * _CH, _CH)
        copies.append(pltpu.async_copy(uemb_hbm.at[uidx_v.at[sl]], urows_v.at[sl], sem))
        copies.append(pltpu.async_copy(memb_hbm.at[midx_v.at[sl]], mrows_v.at[sl], sem))
    for c in copies:
        c.wait()
    pltpu.sync_copy(urows_v, u_out.at[pl.ds(base, _BPW)])
    pltpu.sync_copy(mrows_v, m_out.at[pl.ds(base, _BPW)])


@functools.cache
def _make_gather():
    return pl.kernel(
        _gather_body,
        mesh=plsc.VectorSubcoreMesh(core_axis_name="c", subcore_axis_name="s"),
        out_type=[
            jax.ShapeDtypeStruct((B, D), jnp.float32),
            jax.ShapeDtypeStruct((B, D), jnp.float32),
        ],
        scratch_types=[
            pltpu.VMEM((_BPW,), jnp.int32),
            pltpu.VMEM((_BPW,), jnp.int32),
            pltpu.VMEM((_BPW, D), jnp.float32),
            pltpu.VMEM((_BPW, D), jnp.float32),
            pltpu.SemaphoreType.DMA,
        ],
        compiler_params=pltpu.CompilerParams(use_tc_tiling_on_sc=False),
    )


def _mlp_body(u_ref, m_ref, w1u_ref, w1m_ref, b1_ref, w2_ref, b2_ref,
              w3_ref, b3_ref, out_ref):
    h1 = jnp.dot(u_ref[...], w1u_ref[...], preferred_element_type=jnp.float32)
    h1 = h1 + jnp.dot(m_ref[...], w1m_ref[...], preferred_element_type=jnp.float32)
    h1 = jnp.maximum(h1 + b1_ref[...], 0.0)
    h2 = jnp.dot(h1, w2_ref[...], preferred_element_type=jnp.float32)
    h2 = jnp.maximum(h2 + b2_ref[...], 0.0)
    out_ref[...] = jnp.sum(h2 * w3_ref[...], axis=1, keepdims=True) + b3_ref[...]


@functools.cache
def _make_mlp():
    return pl.pallas_call(
        _mlp_body,
        grid=(B // _BT,),
        in_specs=[
            pl.BlockSpec((_BT, D), lambda i: (i, 0)),
            pl.BlockSpec((_BT, D), lambda i: (i, 0)),
            pl.BlockSpec((D, H1), lambda i: (0, 0)),
            pl.BlockSpec((D, H1), lambda i: (0, 0)),
            pl.BlockSpec((1, H1), lambda i: (0, 0)),
            pl.BlockSpec((H1, H2), lambda i: (0, 0)),
            pl.BlockSpec((1, H2), lambda i: (0, 0)),
            pl.BlockSpec((1, H2), lambda i: (0, 0)),
            pl.BlockSpec((1, 1), lambda i: (0, 0)),
        ],
        out_specs=pl.BlockSpec((_BT, 1), lambda i: (i, 0)),
        out_shape=jax.ShapeDtypeStruct((B, 1), jnp.float32),
        compiler_params=pltpu.CompilerParams(
            dimension_semantics=("arbitrary",),
        ),
    )


def kernel(user_id, movie_id, user_emb, movie_emb, W1, b1, W2, b2, W3, b3):
    u, m = _make_gather()(user_id.astype(jnp.int32), movie_id.astype(jnp.int32),
                          user_emb, movie_emb)
    return _make_mlp()(
        u, m,
        W1[:, :D].T, W1[:, D:].T, b1.reshape(1, H1),
        W2.T, b2.reshape(1, H2),
        W3.reshape(1, H2), b3.reshape(1, 1),
    )


# speedup vs baseline: 1.5800x; 1.5800x over previous
import functools

import jax
import jax.numpy as jnp
from jax import lax
from jax.experimental import pallas as pl
from jax.experimental.pallas import tpu as pltpu
from jax.experimental.pallas import tpu_sc as plsc

B = 16384
D = 64
H1 = 128
H2 = 64

_NC = 2
_NS = 16
_NW = _NC * _NS
_BPW = B // _NW
_CH = 128
_NCH = _BPW // _CH

_BT = 2048


_W = 16
_R = 256


def _gather_body(uid_hbm, mid_hbm, uemb_hbm, memb_hbm, u_out, m_out,
                 uidx_v, midx_v, urows_v, mrows_v, sem):
    wid = lax.axis_index("s") * _NC + lax.axis_index("c")
    base = wid * _BPW
    pltpu.sync_copy(uid_hbm.at[pl.ds(base, _BPW)], uidx_v)
    pltpu.sync_copy(mid_hbm.at[pl.ds(base, _BPW)], midx_v)

    for rnd in range(_BPW // _R):
        def wave(w, carry):
            cu = uidx_v[pl.ds(rnd * _R + w * _W, _W)]
            cm = midx_v[pl.ds(rnd * _R + w * _W, _W)]
            hs = []
            for j in range(_W):
                rr = w * _W + j
                hs.append(pltpu.async_copy(
                    uemb_hbm.at[cu[j]], urows_v.at[rr, pl.ds(0, D)], sem))
                hs.append(pltpu.async_copy(
                    memb_hbm.at[cm[j]], mrows_v.at[rr, pl.ds(0, D)], sem))
            for h in hs:
                h.wait()
            return carry

        lax.fori_loop(0, _R // _W, wave, 0)
        pltpu.sync_copy(urows_v, u_out.at[pl.ds(base + rnd * _R, _R)])
        pltpu.sync_copy(mrows_v, m_out.at[pl.ds(base + rnd * _R, _R)])


@functools.cache
def _make_gather():
    return pl.kernel(
        _gather_body,
        mesh=plsc.VectorSubcoreMesh(core_axis_name="c", subcore_axis_name="s"),
        out_type=[
            jax.ShapeDtypeStruct((B, 2 * D), jnp.float32),
            jax.ShapeDtypeStruct((B, 2 * D), jnp.float32),
        ],
        scratch_types=[
            pltpu.VMEM((_BPW,), jnp.int32),
            pltpu.VMEM((_BPW,), jnp.int32),
            pltpu.VMEM((_R, 2 * D), jnp.float32),
            pltpu.VMEM((_R, 2 * D), jnp.float32),
            pltpu.SemaphoreType.DMA,
        ],
        compiler_params=pltpu.CompilerParams(use_tc_tiling_on_sc=True),
    )


def _mlp_body(u_ref, m_ref, w1u_ref, w1m_ref, b1_ref, w2_ref, b2_ref,
              w3_ref, b3_ref, out_ref):
    h1 = jnp.dot(u_ref[:, :D], w1u_ref[...], preferred_element_type=jnp.float32)
    h1 = h1 + jnp.dot(m_ref[:, :D], w1m_ref[...], preferred_element_type=jnp.float32)
    h1 = jnp.maximum(h1 + b1_ref[...], 0.0)
    h2 = jnp.dot(h1, w2_ref[...], preferred_element_type=jnp.float32)
    h2 = jnp.maximum(h2 + b2_ref[...], 0.0)
    out_ref[...] = jnp.sum(h2 * w3_ref[...], axis=1, keepdims=True) + b3_ref[...]


@functools.cache
def _make_mlp():
    return pl.pallas_call(
        _mlp_body,
        grid=(B // _BT,),
        in_specs=[
            pl.BlockSpec((_BT, 2 * D), lambda i: (i, 0)),
            pl.BlockSpec((_BT, 2 * D), lambda i: (i, 0)),
            pl.BlockSpec((D, H1), lambda i: (0, 0)),
            pl.BlockSpec((D, H1), lambda i: (0, 0)),
            pl.BlockSpec((1, H1), lambda i: (0, 0)),
            pl.BlockSpec((H1, H2), lambda i: (0, 0)),
            pl.BlockSpec((1, H2), lambda i: (0, 0)),
            pl.BlockSpec((1, H2), lambda i: (0, 0)),
            pl.BlockSpec((1, 1), lambda i: (0, 0)),
        ],
        out_specs=pl.BlockSpec((_BT, 1), lambda i: (i, 0)),
        out_shape=jax.ShapeDtypeStruct((B, 1), jnp.float32),
        compiler_params=pltpu.CompilerParams(
            dimension_semantics=("arbitrary",),
        ),
    )


def kernel(user_id, movie_id, user_emb, movie_emb, W1, b1, W2, b2, W3, b3):
    u, m = _make_gather()(user_id.astype(jnp.int32), movie_id.astype(jnp.int32),
                          user_emb, movie_emb)
    return _make_mlp()(
        u, m,
        W1[:, :D].T, W1[:, D:].T, b1.reshape(1, H1),
        W2.T, b2.reshape(1, H2),
        W3.reshape(1, H2), b3.reshape(1, 1),
    )
